# baseline (speedup 1.0000x reference)
import jax
import jax.numpy as jnp
from jax import lax
from jax.experimental import pallas as pl
from jax.experimental.pallas import tpu as pltpu
from jax.experimental.pallas import tpu_sc as plsc

_N = 1600000
_S = 1024
_EPS = 1e-5
_L = 16
_NC = 2
_NS = 16
_NW = _NC * _NS
_RW = _N // _NW
_SUB = 2000
_NSUB = _RW // _SUB
_PAIRS = _NSUB // 2
_VECS = _SUB // _L
_SEGW = _S // 8

_mesh = plsc.VectorSubcoreMesh(core_axis_name="c", subcore_axis_name="s")


def _rsqrt(x):
    i = plsc.bitcast(x, jnp.int32)
    i = jnp.int32(0x5F3759DF) - lax.shift_right_arithmetic(i, 1)
    y = plsc.bitcast(i, jnp.float32)
    hx = 0.5 * x
    for _ in range(2):
        y = y * (1.5 - hx * y * y)
    return y


def _wid():
    return lax.axis_index("s") * _NC + lax.axis_index("c")


def _start_in(srcs, bufs, sem, r0):
    for src, buf in zip(srcs, bufs):
        pltpu.async_copy(src.at[pl.ds(r0, _SUB)], buf, sem)


def _wait_in(srcs, bufs, sem):
    for src, buf in zip(srcs, bufs):
        pltpu.make_async_copy(src.at[pl.ds(0, _SUB)], buf, sem).wait()


def _in_ring(srcs, bufs_a, bufs_b, sem_a, sem_b, row0, process):
    _start_in(srcs, bufs_a, sem_a, row0)

    def body(t, _):
        j0 = 2 * t
        _wait_in(srcs, bufs_a, sem_a)
        _start_in(srcs, bufs_b, sem_b, row0 + (j0 + 1) * _SUB)
        process(bufs_a, j0)

        @pl.when(j0 + 2 < _NSUB)
        def _():
            _start_in(srcs, bufs_a, sem_a, row0 + (j0 + 2) * _SUB)

        _wait_in(srcs, bufs_b, sem_b)
        process(bufs_b, j0 + 1)
        return 0

    lax.fori_loop(0, _PAIRS, body, 0)
    if _NSUB % 2:
        _wait_in(srcs, bufs_a, sem_a)
        process(bufs_a, _NSUB - 1)


def _partials_body(xs_hbm, ys_hbm, zs_hbm, batch_hbm, out_hbm,
                   xa_v, ya_v, za_v, ia_v, xb_v, yb_v, zb_v, ib_v,
                   sums_v, cnts_v, stage_v, sem_a, sem_b):
    wid = _wid()
    iota = lax.iota(jnp.int32, _L)
    zeros = jnp.zeros((_L,), jnp.float32)
    ones = jnp.ones((_L,), jnp.float32)
    row0 = wid * _RW

    @plsc.parallel_loop(0, _S, unroll=8)
    def _(i):
        sums_v[pl.ds(i * _L, _L)] = zeros
        cnts_v[pl.ds(i * _L, _L)] = zeros

    def process(bufs, j):
        xs_v, ys_v, zs_v, ids_v = bufs

        @plsc.parallel_loop(0, _VECS, unroll=16)
        def _(k):
            sl = pl.ds(k * _L, _L)
            x = xs_v[sl]
            y = ys_v[sl]
            z = zs_v[sl]
            ss = x * x + y * y + z * z
            nrm = ss * _rsqrt(ss)
            fidx = ids_v[sl] * _L + iota
            plsc.addupdate_scatter(sums_v, [fidx], nrm)
            plsc.addupdate_scatter(cnts_v, [fidx], ones)

    srcs = (xs_hbm, ys_hbm, zs_hbm, batch_hbm)
    _in_ring(srcs, (xa_v, ya_v, za_v, ia_v), (xb_v, yb_v, zb_v, ib_v),
             sem_a, sem_b, row0, process)

    @plsc.parallel_loop(0, _S // _L, unroll=2)
    def _(s):
        base = (s * _L + iota) * _L
        a = zeros
        c = zeros
        for j in range(_L):
            a = a + plsc.load_gather(sums_v, [base + j])
            c = c + plsc.load_gather(cnts_v, [base + j])
        stage_v[pl.ds(s * _L, _L)] = a
        stage_v[pl.ds(_S + s * _L, _L)] = c

    pltpu.sync_copy(stage_v, out_hbm.at[wid])


def _apply_body(xs_hbm, ys_hbm, zs_hbm, batch_hbm, part_hbm, w_hbm,
                xo_hbm, yo_hbm, zo_hbm,
                xa_v, ya_v, za_v, ia_v, xb_v, yb_v, zb_v, ib_v,
                oxa_v, oya_v, oza_v, oxb_v, oyb_v, ozb_v,
                folds_v, foldc_v, invq_v, inv_v, w_v, inv_sh,
                sem_a, sem_b, sem_oa, sem_ob):
    wid = _wid()
    q = lax.axis_index("s")
    row0 = wid * _RW

    @pl.when(q < 8)
    def _():
        pltpu.sync_copy(part_hbm.at[:, pl.ds(q * _SEGW, _SEGW)], folds_v)
        pltpu.sync_copy(part_hbm.at[:, pl.ds(_S + q * _SEGW, _SEGW)], foldc_v)
        pltpu.sync_copy(w_hbm, w_v)
        wvec = w_v[...]

        for b in range(_SEGW // _L):
            def fold_one(w, carry, _b=b):
                a, c = carry
                a = a + folds_v[w, pl.ds(_b * _L, _L)]
                c = c + foldc_v[w, pl.ds(_b * _L, _L)]
                return (a, c)

            a, c = lax.fori_loop(
                0, _NW, fold_one,
                (jnp.zeros((_L,), jnp.float32), jnp.zeros((_L,), jnp.float32)))
            mean = a / jnp.maximum(c, 1.0)
            invq_v[pl.ds(b * _L, _L)] = wvec / (mean + _EPS)

        pltpu.sync_copy(invq_v, inv_sh.at[pl.ds(q * _SEGW, _SEGW)])

    plsc.subcore_barrier()
    pltpu.sync_copy(inv_sh, inv_v)

    outs = ((oxa_v, oya_v, oza_v, sem_oa), (oxb_v, oyb_v, ozb_v, sem_ob))
    dsts = (xo_hbm, yo_hbm, zo_hbm)

    for ox_v, oy_v, oz_v, sem_o in outs:
        for dst, o in zip(dsts, (ox_v, oy_v, oz_v)):
            pltpu.async_copy(dst.at[pl.ds(0, _SUB)], o, sem_o)

    def process(bufs, j):
        xs_v, ys_v, zs_v, ids_v = bufs
        slot = 0 if bufs[0] is xa_v else 1
        ox_v, oy_v, oz_v, sem_o = outs[slot]
        for dst, o in zip(dsts, (ox_v, oy_v, oz_v)):
            pltpu.make_async_copy(dst.at[pl.ds(0, _SUB)], o, sem_o).wait()

        @plsc.parallel_loop(0, _VECS, unroll=16)
        def _(k):
            sl = pl.ds(k * _L, _L)
            sc = plsc.load_gather(inv_v, [ids_v[sl]])
            ox_v[sl] = xs_v[sl] * sc
            oy_v[sl] = ys_v[sl] * sc
            oz_v[sl] = zs_v[sl] * sc

        for dst, o in zip(dsts, (ox_v, oy_v, oz_v)):
            pltpu.async_copy(o, dst.at[pl.ds(row0 + j * _SUB, _SUB)], sem_o)

    srcs = (xs_hbm, ys_hbm, zs_hbm, batch_hbm)
    _in_ring(srcs, (xa_v, ya_v, za_v, ia_v), (xb_v, yb_v, zb_v, ib_v),
             sem_a, sem_b, row0, process)

    for ox_v, oy_v, oz_v, sem_o in outs:
        for dst, o in zip(dsts, (ox_v, oy_v, oz_v)):
            pltpu.make_async_copy(dst.at[pl.ds(0, _SUB)], o, sem_o).wait()


_f32vec = jax.ShapeDtypeStruct((_N,), jnp.float32)


def _vmem_f32(n):
    return pltpu.VMEM((n,), jnp.float32)


_k1 = pl.kernel(
    _partials_body,
    out_type=jax.ShapeDtypeStruct((_NW, 2 * _S), jnp.float32),
    mesh=_mesh,
    compiler_params=pltpu.CompilerParams(needs_layout_passes=False),
    scratch_types=[
        _vmem_f32(_SUB), _vmem_f32(_SUB), _vmem_f32(_SUB),
        pltpu.VMEM((_SUB,), jnp.int32),
        _vmem_f32(_SUB), _vmem_f32(_SUB), _vmem_f32(_SUB),
        pltpu.VMEM((_SUB,), jnp.int32),
        _vmem_f32(_S * _L),
        _vmem_f32(_S * _L),
        _vmem_f32(2 * _S),
        pltpu.SemaphoreType.DMA,
        pltpu.SemaphoreType.DMA,
    ],
)

_k2 = pl.kernel(
    _apply_body,
    out_type=(_f32vec, _f32vec, _f32vec),
    mesh=_mesh,
    compiler_params=pltpu.CompilerParams(needs_layout_passes=False),
    scratch_types=[
        _vmem_f32(_SUB), _vmem_f32(_SUB), _vmem_f32(_SUB),
        pltpu.VMEM((_SUB,), jnp.int32),
        _vmem_f32(_SUB), _vmem_f32(_SUB), _vmem_f32(_SUB),
        pltpu.VMEM((_SUB,), jnp.int32),
        _vmem_f32(_SUB), _vmem_f32(_SUB), _vmem_f32(_SUB),
        _vmem_f32(_SUB), _vmem_f32(_SUB), _vmem_f32(_SUB),
        pltpu.VMEM((_NW, _SEGW), jnp.float32),
        pltpu.VMEM((_NW, _SEGW), jnp.float32),
        _vmem_f32(_SEGW),
        _vmem_f32(_S),
        _vmem_f32(_L),
        pltpu.VMEM_SHARED((_S,), jnp.float32),
        pltpu.SemaphoreType.DMA,
        pltpu.SemaphoreType.DMA,
        pltpu.SemaphoreType.DMA,
        pltpu.SemaphoreType.DMA,
    ],
)


def kernel(pos, batch, weight):
    xs = pos[:, 0]
    ys = pos[:, 1]
    zs = pos[:, 2]
    w16 = jnp.broadcast_to(weight.reshape(()), (_L,))
    partials = _k1(xs, ys, zs, batch)
    xo, yo, zo = _k2(xs, ys, zs, batch, partials, w16)
    col = lax.broadcasted_iota(jnp.int32, (1, 3), 1)
    out = jnp.where(col == 0, xo[:, None],
                    jnp.where(col == 1, yo[:, None], zo[:, None]))
    return out


# speedup vs baseline: 1.0447x; 1.0447x over previous
import jax
import jax.numpy as jnp
from jax import lax
from jax.experimental import pallas as pl
from jax.experimental.pallas import tpu as pltpu
from jax.experimental.pallas import tpu_sc as plsc

_N = 1600000
_S = 1024
_EPS = 1e-5
_L = 16
_NC = 2
_NS = 16
_NW = _NC * _NS
_RW = _N // _NW
_SUB = 2000
_NSUB = _RW // _SUB
_PAIRS = _NSUB // 2
_VECS = _SUB // _L
_SEGW = _S // 8

_mesh = plsc.VectorSubcoreMesh(core_axis_name="c", subcore_axis_name="s")


def _rsqrt(x):
    i = plsc.bitcast(x, jnp.int32)
    i = jnp.int32(0x5F3759DF) - lax.shift_right_arithmetic(i, 1)
    y = plsc.bitcast(i, jnp.float32)
    hx = 0.5 * x
    for _ in range(2):
        y = y * (1.5 - hx * y * y)
    return y


def _wid():
    return lax.axis_index("s") * _NC + lax.axis_index("c")


def _start_in(srcs, bufs, sem, r0):
    for src, buf in zip(srcs, bufs):
        pltpu.async_copy(src.at[pl.ds(r0, _SUB)], buf, sem)


def _wait_in(srcs, bufs, sem):
    for src, buf in zip(srcs, bufs):
        pltpu.make_async_copy(src.at[pl.ds(0, _SUB)], buf, sem).wait()


def _in_ring(srcs, bufs_a, bufs_b, sem_a, sem_b, row0, process):
    _start_in(srcs, bufs_a, sem_a, row0)

    def body(t, _):
        j0 = 2 * t
        _wait_in(srcs, bufs_a, sem_a)
        _start_in(srcs, bufs_b, sem_b, row0 + (j0 + 1) * _SUB)
        process(bufs_a, j0)

        @pl.when(j0 + 2 < _NSUB)
        def _():
            _start_in(srcs, bufs_a, sem_a, row0 + (j0 + 2) * _SUB)

        _wait_in(srcs, bufs_b, sem_b)
        process(bufs_b, j0 + 1)
        return 0

    lax.fori_loop(0, _PAIRS, body, 0)
    if _NSUB % 2:
        _wait_in(srcs, bufs_a, sem_a)
        process(bufs_a, _NSUB - 1)


def _partials_body(xs_hbm, ys_hbm, zs_hbm, batch_hbm, out_hbm,
                   xa_v, ya_v, za_v, ia_v, xb_v, yb_v, zb_v, ib_v,
                   sums_v, cnts_v, stage_v, sem_a, sem_b):
    wid = _wid()
    iota = lax.iota(jnp.int32, _L)
    zeros = jnp.zeros((_L,), jnp.float32)
    ones = jnp.ones((_L,), jnp.float32)
    row0 = wid * _RW

    @plsc.parallel_loop(0, _S, unroll=8)
    def _(i):
        sums_v[pl.ds(i * _L, _L)] = zeros
        cnts_v[pl.ds(i * _L, _L)] = zeros

    def process(bufs, j):
        xs_v, ys_v, zs_v, ids_v = bufs

        @plsc.parallel_loop(0, _VECS, unroll=8)
        def _(k):
            sl = pl.ds(k * _L, _L)
            x = xs_v[sl]
            y = ys_v[sl]
            z = zs_v[sl]
            ss = x * x + y * y + z * z
            nrm = ss * _rsqrt(ss)
            fidx = ids_v[sl] * _L + iota
            plsc.addupdate_scatter(sums_v, [fidx], nrm)
            plsc.addupdate_scatter(cnts_v, [fidx], ones)

    srcs = (xs_hbm, ys_hbm, zs_hbm, batch_hbm)
    _in_ring(srcs, (xa_v, ya_v, za_v, ia_v), (xb_v, yb_v, zb_v, ib_v),
             sem_a, sem_b, row0, process)

    @plsc.parallel_loop(0, _S // _L, unroll=2)
    def _(s):
        base = (s * _L + iota) * _L
        a = zeros
        c = zeros
        for j in range(_L):
            a = a + plsc.load_gather(sums_v, [base + j])
            c = c + plsc.load_gather(cnts_v, [base + j])
        stage_v[pl.ds(s * _L, _L)] = a
        stage_v[pl.ds(_S + s * _L, _L)] = c

    pltpu.sync_copy(stage_v, out_hbm.at[wid])


def _apply_body(xs_hbm, ys_hbm, zs_hbm, batch_hbm, part_hbm, w_hbm,
                xo_hbm, yo_hbm, zo_hbm,
                xa_v, ya_v, za_v, ia_v, xb_v, yb_v, zb_v, ib_v,
                oxa_v, oya_v, oza_v, oxb_v, oyb_v, ozb_v,
                folds_v, foldc_v, invq_v, inv_v, w_v, inv_sh,
                sem_a, sem_b, sem_oa, sem_ob):
    wid = _wid()
    q = lax.axis_index("s")
    row0 = wid * _RW

    @pl.when(q < 8)
    def _():
        pltpu.sync_copy(part_hbm.at[:, pl.ds(q * _SEGW, _SEGW)], folds_v)
        pltpu.sync_copy(part_hbm.at[:, pl.ds(_S + q * _SEGW, _SEGW)], foldc_v)
        pltpu.sync_copy(w_hbm, w_v)
        wvec = w_v[...]

        for b in range(_SEGW // _L):
            def fold_one(w, carry, _b=b):
                a, c = carry
                a = a + folds_v[w, pl.ds(_b * _L, _L)]
                c = c + foldc_v[w, pl.ds(_b * _L, _L)]
                return (a, c)

            a, c = lax.fori_loop(
                0, _NW, fold_one,
                (jnp.zeros((_L,), jnp.float32), jnp.zeros((_L,), jnp.float32)))
            mean = a / jnp.maximum(c, 1.0)
            invq_v[pl.ds(b * _L, _L)] = wvec / (mean + _EPS)

        pltpu.sync_copy(invq_v, inv_sh.at[pl.ds(q * _SEGW, _SEGW)])

    plsc.subcore_barrier()
    pltpu.sync_copy(inv_sh, inv_v)

    outs = ((oxa_v, oya_v, oza_v, sem_oa), (oxb_v, oyb_v, ozb_v, sem_ob))
    dsts = (xo_hbm, yo_hbm, zo_hbm)

    for ox_v, oy_v, oz_v, sem_o in outs:
        for dst, o in zip(dsts, (ox_v, oy_v, oz_v)):
            pltpu.async_copy(dst.at[pl.ds(0, _SUB)], o, sem_o)

    def process(bufs, j):
        xs_v, ys_v, zs_v, ids_v = bufs
        slot = 0 if bufs[0] is xa_v else 1
        ox_v, oy_v, oz_v, sem_o = outs[slot]
        for dst, o in zip(dsts, (ox_v, oy_v, oz_v)):
            pltpu.make_async_copy(dst.at[pl.ds(0, _SUB)], o, sem_o).wait()

        @plsc.parallel_loop(0, _VECS, unroll=8)
        def _(k):
            sl = pl.ds(k * _L, _L)
            sc = plsc.load_gather(inv_v, [ids_v[sl]])
            ox_v[sl] = xs_v[sl] * sc
            oy_v[sl] = ys_v[sl] * sc
            oz_v[sl] = zs_v[sl] * sc

        for dst, o in zip(dsts, (ox_v, oy_v, oz_v)):
            pltpu.async_copy(o, dst.at[pl.ds(row0 + j * _SUB, _SUB)], sem_o)

    srcs = (xs_hbm, ys_hbm, zs_hbm, batch_hbm)
    _in_ring(srcs, (xa_v, ya_v, za_v, ia_v), (xb_v, yb_v, zb_v, ib_v),
             sem_a, sem_b, row0, process)

    for ox_v, oy_v, oz_v, sem_o in outs:
        for dst, o in zip(dsts, (ox_v, oy_v, oz_v)):
            pltpu.make_async_copy(dst.at[pl.ds(0, _SUB)], o, sem_o).wait()


_f32vec = jax.ShapeDtypeStruct((_N,), jnp.float32)


def _vmem_f32(n):
    return pltpu.VMEM((n,), jnp.float32)


_k1 = pl.kernel(
    _partials_body,
    out_type=jax.ShapeDtypeStruct((_NW, 2 * _S), jnp.float32),
    mesh=_mesh,
    compiler_params=pltpu.CompilerParams(needs_layout_passes=False),
    scratch_types=[
        _vmem_f32(_SUB), _vmem_f32(_SUB), _vmem_f32(_SUB),
        pltpu.VMEM((_SUB,), jnp.int32),
        _vmem_f32(_SUB), _vmem_f32(_SUB), _vmem_f32(_SUB),
        pltpu.VMEM((_SUB,), jnp.int32),
        _vmem_f32(_S * _L),
        _vmem_f32(_S * _L),
        _vmem_f32(2 * _S),
        pltpu.SemaphoreType.DMA,
        pltpu.SemaphoreType.DMA,
    ],
)

_k2 = pl.kernel(
    _apply_body,
    out_type=(_f32vec, _f32vec, _f32vec),
    mesh=_mesh,
    compiler_params=pltpu.CompilerParams(needs_layout_passes=False),
    scratch_types=[
        _vmem_f32(_SUB), _vmem_f32(_SUB), _vmem_f32(_SUB),
        pltpu.VMEM((_SUB,), jnp.int32),
        _vmem_f32(_SUB), _vmem_f32(_SUB), _vmem_f32(_SUB),
        pltpu.VMEM((_SUB,), jnp.int32),
        _vmem_f32(_SUB), _vmem_f32(_SUB), _vmem_f32(_SUB),
        _vmem_f32(_SUB), _vmem_f32(_SUB), _vmem_f32(_SUB),
        pltpu.VMEM((_NW, _SEGW), jnp.float32),
        pltpu.VMEM((_NW, _SEGW), jnp.float32),
        _vmem_f32(_SEGW),
        _vmem_f32(_S),
        _vmem_f32(_L),
        pltpu.VMEM_SHARED((_S,), jnp.float32),
        pltpu.SemaphoreType.DMA,
        pltpu.SemaphoreType.DMA,
        pltpu.SemaphoreType.DMA,
        pltpu.SemaphoreType.DMA,
    ],
)


def kernel(pos, batch, weight):
    xs = pos[:, 0]
    ys = pos[:, 1]
    zs = pos[:, 2]
    w16 = jnp.broadcast_to(weight.reshape(()), (_L,))
    partials = _k1(xs, ys, zs, batch)
    xo, yo, zo = _k2(xs, ys, zs, batch, partials, w16)
    col = lax.broadcasted_iota(jnp.int32, (1, 3), 1)
    out = jnp.where(col == 0, xo[:, None],
                    jnp.where(col == 1, yo[:, None], zo[:, None]))
    return out
